# baseline (speedup 1.0000x reference)
import functools

import jax
import jax.numpy as jnp
from jax import lax
from jax.experimental import pallas as pl
from jax.experimental.pallas import tpu as pltpu
from jax.experimental.pallas import tpu_sc as plsc

N_ATOMS = 50000
N_BONDS = 200000
MAX_NB = 6
HIDDEN = 128
PACKED = HIDDEN // 2
DEPTH = 4
ATOM_FDIM = 144
IN_BOND = ATOM_FDIM + 14
N_MOLS = 2000
ATOMS_PER_MOL = 25

NUM_SC_CORES = 2
NUM_SC_SUBCORES = 16
NUM_WORKERS = NUM_SC_CORES * NUM_SC_SUBCORES


def _make_gather_sum(n_pad, chunk, steps, table_rows):
  assert NUM_WORKERS * steps * chunk == n_pad
  assert chunk % 8 == 0 and chunk <= 128
  assert steps >= 3

  mesh = plsc.VectorSubcoreMesh(core_axis_name="c", subcore_axis_name="s")

  @functools.partial(
      pl.kernel,
      out_type=jax.ShapeDtypeStruct((n_pad, PACKED), jnp.int32),
      mesh=mesh,
      compiler_params=pltpu.CompilerParams(
          needs_layout_passes=False, use_tc_tiling_on_sc=False
      ),
      scratch_types=[
          pltpu.VMEM((2, MAX_NB, chunk), jnp.int32),
          pltpu.VMEM((2, MAX_NB, chunk, PACKED), jnp.int32),
          pltpu.SemaphoreType.DMA((2,)),
          pltpu.SemaphoreType.DMA((2,)),
          pltpu.SemaphoreType.DMA((2,)),
      ],
  )
  def gather_sum(table_hbm, idx_hbm, out_hbm, idx_v, buf_v, isem, gsem, osem):
    s_id = lax.axis_index("s")
    c_id = lax.axis_index("c")
    step0 = (s_id * NUM_SC_CORES + c_id) * steps
    n = steps

    def idx_cp(t, p):
      pltpu.async_copy(idx_hbm.at[step0 + t], idx_v.at[p], isem.at[p])

    def idx_wait(t, p):
      pltpu.make_async_copy(idx_hbm.at[step0 + t], idx_v.at[p],
                            isem.at[p]).wait()

    def fire_gathers(p):
      for k in range(MAX_NB):
        pltpu.async_copy(table_hbm.at[idx_v.at[p, k]], buf_v.at[p, k],
                         gsem.at[p])

    def wait_gathers(p):
      for _ in range(MAX_NB):
        pltpu.make_async_copy(table_hbm.at[idx_v.at[p, 0]], buf_v.at[p, 0],
                              gsem.at[p]).wait()

    def fire_out(t, p):
      pltpu.async_copy(buf_v.at[p, 0],
                       out_hbm.at[pl.ds((step0 + t) * chunk, chunk)],
                       osem.at[p])

    def wait_out(t, p):
      pltpu.make_async_copy(buf_v.at[p, 0],
                            out_hbm.at[pl.ds((step0 + t) * chunk, chunk)],
                            osem.at[p]).wait()

    def sum_slot(p):
      def row_body(r, carry):
        for cg in range(PACKED // 16):
          sl = pl.ds(cg * 16, 16)
          a = [
              plsc.bitcast(buf_v[p, k, r, sl], jnp.bfloat16)
              for k in range(MAX_NB)
          ]
          s = ((a[0] + a[1]) + (a[2] + a[3])) + (a[4] + a[5])
          buf_v[p, 0, r, sl] = plsc.bitcast(s, jnp.int32)
        return carry

      lax.fori_loop(0, chunk, row_body, 0)

    idx_cp(0, 0)
    idx_wait(0, 0)
    fire_gathers(0)
    idx_cp(1, 1)

    wait_gathers(0)
    idx_wait(1, 1)
    fire_gathers(1)
    idx_cp(2, 0)
    sum_slot(0)
    fire_out(0, 0)

    def body(t, carry):
      p = lax.rem(t, 2)
      q = 1 - p
      wait_gathers(p)
      idx_wait(t + 1, q)
      wait_out(t - 1, q)
      fire_gathers(q)
      tn = jnp.minimum(t + 2, n - 1)
      idx_cp(tn, p)
      sum_slot(p)
      fire_out(t, p)
      return carry

    lax.fori_loop(1, n - 1, body, 0)

    p = lax.rem(n - 1, 2)
    q = 1 - p
    wait_gathers(p)
    wait_out(n - 2, q)
    sum_slot(p)
    fire_out(n - 1, p)
    wait_out(n - 1, p)
    idx_wait(n - 1, lax.rem(n, 2))

  return gather_sum


def _pad_indices(graph, n_pad, chunk):
  n = graph.shape[0]
  g = jnp.pad(graph.astype(jnp.int32), ((0, n_pad - n), (0, 0)))
  return g.T.reshape(MAX_NB, n_pad // chunk, chunk).transpose(1, 0, 2)


def _pack_pair(a_even, a_odd):
  ue = lax.bitcast_convert_type(a_even, jnp.uint32)
  ue = ue + jnp.uint32(0x7FFF) + ((ue >> 16) & jnp.uint32(1))
  uo = lax.bitcast_convert_type(a_odd, jnp.uint32)
  uo = uo + jnp.uint32(0x7FFF) + ((uo >> 16) & jnp.uint32(1))
  return lax.bitcast_convert_type(
      (uo & jnp.uint32(0xFFFF0000)) | (ue >> 16), jnp.int32
  )


def _unpack_pair(x):
  u = lax.bitcast_convert_type(x, jnp.uint32)
  lo_f = lax.bitcast_convert_type(u << 16, jnp.float32)
  hi_f = lax.bitcast_convert_type(u & jnp.uint32(0xFFFF0000), jnp.float32)
  return lo_f, hi_f


def _tc_input_proj(fbonds, W_i_e, W_i_o):
  blk = 2000

  def body(x_ref, we_ref, wo_ref, be_ref, bo_ref, m_ref):
    x = x_ref[...]
    b_e = jnp.dot(x, we_ref[...], preferred_element_type=jnp.float32)
    b_o = jnp.dot(x, wo_ref[...], preferred_element_type=jnp.float32)
    be_ref[...] = b_e
    bo_ref[...] = b_o
    m_ref[...] = _pack_pair(jnp.maximum(b_e, 0.0), jnp.maximum(b_o, 0.0))

  return pl.pallas_call(
      body,
      grid=(N_BONDS // blk,),
      in_specs=[
          pl.BlockSpec((blk, IN_BOND), lambda i: (i, 0)),
          pl.BlockSpec((IN_BOND, PACKED), lambda i: (0, 0)),
          pl.BlockSpec((IN_BOND, PACKED), lambda i: (0, 0)),
      ],
      out_specs=[
          pl.BlockSpec((blk, PACKED), lambda i: (i, 0)),
          pl.BlockSpec((blk, PACKED), lambda i: (i, 0)),
          pl.BlockSpec((blk, PACKED), lambda i: (i, 0)),
      ],
      out_shape=[
          jax.ShapeDtypeStruct((N_BONDS, PACKED), jnp.float32),
          jax.ShapeDtypeStruct((N_BONDS, PACKED), jnp.float32),
          jax.ShapeDtypeStruct((N_BONDS, PACKED), jnp.int32),
      ],
  )(fbonds, W_i_e, W_i_o)


def _tc_msg_update(binput_e, binput_o, nei_padded, Whee, Whoe, Wheo, Whoo):
  blk = 2000

  def body(be_ref, bo_ref, n_ref, wee_ref, woe_ref, weo_ref, woo_ref, o_ref):
    lo_f, hi_f = _unpack_pair(n_ref[...])
    t_e = (
        jnp.dot(lo_f, wee_ref[...], preferred_element_type=jnp.float32)
        + jnp.dot(hi_f, woe_ref[...], preferred_element_type=jnp.float32)
    )
    t_o = (
        jnp.dot(lo_f, weo_ref[...], preferred_element_type=jnp.float32)
        + jnp.dot(hi_f, woo_ref[...], preferred_element_type=jnp.float32)
    )
    m_e = jnp.maximum(be_ref[...] + t_e, 0.0)
    m_o = jnp.maximum(bo_ref[...] + t_o, 0.0)
    o_ref[...] = _pack_pair(m_e, m_o)

  w_spec = pl.BlockSpec((PACKED, PACKED), lambda i: (0, 0))
  return pl.pallas_call(
      body,
      grid=(N_BONDS // blk,),
      in_specs=[
          pl.BlockSpec((blk, PACKED), lambda i: (i, 0)),
          pl.BlockSpec((blk, PACKED), lambda i: (i, 0)),
          pl.BlockSpec((blk, PACKED), lambda i: (i, 0)),
          w_spec, w_spec, w_spec, w_spec,
      ],
      out_specs=pl.BlockSpec((blk, PACKED), lambda i: (i, 0)),
      out_shape=jax.ShapeDtypeStruct((N_BONDS, PACKED), jnp.int32),
  )(binput_e, binput_o, nei_padded, Whee, Whoe, Wheo, Whoo)


def _tc_readout(fatoms, anei, W_oa, W_on_e, W_on_o, b_o, seg):
  blk = 1000
  mols_per_blk = blk // ATOMS_PER_MOL

  def body(fa_ref, an_ref, woa_ref, we_ref, wo_ref, bo_ref, seg_ref, o_ref):
    lo_f, hi_f = _unpack_pair(an_ref[...])
    h = (
        jnp.dot(fa_ref[...], woa_ref[...], preferred_element_type=jnp.float32)
        + jnp.dot(lo_f, we_ref[...], preferred_element_type=jnp.float32)
        + jnp.dot(hi_f, wo_ref[...], preferred_element_type=jnp.float32)
        + bo_ref[...]
    )
    h = jnp.maximum(h, 0.0)
    o_ref[...] = jnp.dot(seg_ref[...], h, preferred_element_type=jnp.float32)

  return pl.pallas_call(
      body,
      grid=(N_ATOMS // blk,),
      in_specs=[
          pl.BlockSpec((blk, ATOM_FDIM), lambda i: (i, 0)),
          pl.BlockSpec((blk, PACKED), lambda i: (i, 0)),
          pl.BlockSpec((ATOM_FDIM, HIDDEN), lambda i: (0, 0)),
          pl.BlockSpec((PACKED, HIDDEN), lambda i: (0, 0)),
          pl.BlockSpec((PACKED, HIDDEN), lambda i: (0, 0)),
          pl.BlockSpec((1, HIDDEN), lambda i: (0, 0)),
          pl.BlockSpec((mols_per_blk, blk), lambda i: (0, 0)),
      ],
      out_specs=pl.BlockSpec((mols_per_blk, HIDDEN), lambda i: (i, 0)),
      out_shape=jax.ShapeDtypeStruct((N_MOLS, HIDDEN), jnp.float32),
  )(fatoms, anei, W_oa, W_on_e, W_on_o, b_o, seg)


BOND_CHUNK = 128
BOND_STEPS = 49
BOND_PAD = NUM_WORKERS * BOND_STEPS * BOND_CHUNK

ATOM_CHUNK = 112
ATOM_STEPS = 14
ATOM_PAD = NUM_WORKERS * ATOM_STEPS * ATOM_CHUNK

_bond_gather = _make_gather_sum(BOND_PAD, BOND_CHUNK, BOND_STEPS, N_BONDS)
_atom_gather = _make_gather_sum(ATOM_PAD, ATOM_CHUNK, ATOM_STEPS, N_BONDS)


def kernel(fatoms, fbonds, agraph, bgraph, W_i, W_h, W_o, b_o):
  bidx = _pad_indices(bgraph, BOND_PAD, BOND_CHUNK)
  aidx = _pad_indices(agraph, ATOM_PAD, ATOM_CHUNK)

  binput_e, binput_o, message = _tc_input_proj(
      fbonds, W_i[:, 0::2], W_i[:, 1::2]
  )

  Whee = W_h[0::2, 0::2]
  Whoe = W_h[1::2, 0::2]
  Wheo = W_h[0::2, 1::2]
  Whoo = W_h[1::2, 1::2]
  for _ in range(DEPTH - 1):
    nei = _bond_gather(message, bidx)
    message = _tc_msg_update(binput_e, binput_o, nei, Whee, Whoe, Wheo, Whoo)

  anei = _atom_gather(message, aidx)

  W_oa = W_o[:ATOM_FDIM]
  W_on = W_o[ATOM_FDIM:]
  blk = 1000
  mols_per_blk = blk // ATOMS_PER_MOL
  seg = jnp.kron(
      jnp.eye(mols_per_blk, dtype=jnp.float32),
      jnp.full((1, ATOMS_PER_MOL), 1.0 / ATOMS_PER_MOL, dtype=jnp.float32),
  )
  return _tc_readout(
      fatoms, anei, W_oa, W_on[0::2], W_on[1::2], b_o.reshape(1, HIDDEN), seg
  )


# speedup vs baseline: 1.1901x; 1.1901x over previous
import functools

import jax
import jax.numpy as jnp
from jax import lax
from jax.experimental import pallas as pl
from jax.experimental.pallas import tpu as pltpu
from jax.experimental.pallas import tpu_sc as plsc

N_ATOMS = 50000
N_BONDS = 200000
MAX_NB = 6
HIDDEN = 128
DEPTH = 4
ATOM_FDIM = 144
IN_BOND = ATOM_FDIM + 14
N_MOLS = 2000
ATOMS_PER_MOL = 25

NUM_SC_CORES = 2
NUM_SC_SUBCORES = 16
NUM_WORKERS = NUM_SC_CORES * NUM_SC_SUBCORES


NBUF = 4


def _make_gather_sum(n_pad, chunk, steps, table_rows):
  assert NUM_WORKERS * steps * chunk == n_pad
  assert chunk % 8 == 0 and chunk <= 128

  mesh = plsc.VectorSubcoreMesh(core_axis_name="c", subcore_axis_name="s")

  @functools.partial(
      pl.kernel,
      out_type=jax.ShapeDtypeStruct((n_pad, HIDDEN), jnp.float32),
      mesh=mesh,
      scratch_types=[
          pltpu.VMEM((NBUF, MAX_NB, chunk), jnp.int32),
          pltpu.VMEM((NBUF, chunk, HIDDEN), jnp.float32),
          pltpu.SemaphoreType.DMA((NBUF,)),
          pltpu.SemaphoreType.DMA((NBUF,)),
          pltpu.SemaphoreType.DMA((NBUF,)),
          pltpu.SemaphoreType.DMA((NBUF,)),
      ],
  )
  def gather_sum(table_hbm, idx_hbm, out_hbm, idx_v, acc_v, isem, gsem, asem,
                 osem):
    s_id = lax.axis_index("s")
    c_id = lax.axis_index("c")
    step0 = (s_id * NUM_SC_CORES + c_id) * steps
    my_steps = steps

    def idx_cp(t, p):
      return pltpu.async_copy(idx_hbm.at[step0 + t], idx_v.at[p], isem.at[p])

    def stage_a(t, p):
      pltpu.make_async_copy(idx_hbm.at[step0 + t], idx_v.at[p],
                            isem.at[p]).wait()
      pltpu.async_copy(table_hbm.at[idx_v.at[p, 0]], acc_v.at[p], gsem.at[p])

    def stage_b(t, p):
      pltpu.make_async_copy(table_hbm.at[idx_v.at[p, 0]], acc_v.at[p],
                            gsem.at[p]).wait()
      for k in range(1, MAX_NB):
        pltpu.async_copy(table_hbm.at[idx_v.at[p, k]], acc_v.at[p], asem.at[p],
                         add=True)

    def stage_c(t, p):
      for _ in range(MAX_NB - 1):
        pltpu.make_async_copy(table_hbm.at[idx_v.at[p, 1]], acc_v.at[p],
                              asem.at[p]).wait()
      pltpu.async_copy(acc_v.at[p],
                       out_hbm.at[pl.ds((step0 + t) * chunk, chunk)],
                       osem.at[p])

    def stage_d(t, p):
      pltpu.make_async_copy(acc_v.at[p],
                            out_hbm.at[pl.ds((step0 + t) * chunk, chunk)],
                            osem.at[p]).wait()

    for t in range(NBUF):
      idx_cp(t, t)
    stage_a(0, 0)
    stage_b(0, 0)
    stage_a(1, 1)
    stage_c(0, 0)
    stage_b(1, 1)
    stage_a(2, 2)

    def body(t, carry):
      p = lax.rem(t, NBUF)
      pm1 = lax.rem(t - 1, NBUF)
      pm2 = lax.rem(t - 2, NBUF)
      pm3 = lax.rem(t - 3, NBUF)
      stage_a(t, p)
      stage_b(t - 1, pm1)
      stage_c(t - 2, pm2)
      stage_d(t - 3, pm3)
      idx_cp(t + 1, pm3)
      return carry

    lax.fori_loop(3, my_steps - 1, body, 0)

    t = my_steps - 1
    stage_a(t, lax.rem(t, NBUF))
    stage_b(t - 1, lax.rem(t - 1, NBUF))
    stage_c(t - 2, lax.rem(t - 2, NBUF))
    stage_d(t - 3, lax.rem(t - 3, NBUF))
    stage_b(t, lax.rem(t, NBUF))
    stage_c(t - 1, lax.rem(t - 1, NBUF))
    stage_d(t - 2, lax.rem(t - 2, NBUF))
    stage_c(t, lax.rem(t, NBUF))
    stage_d(t - 1, lax.rem(t - 1, NBUF))
    stage_d(t, lax.rem(t, NBUF))

  return gather_sum


def _pad_indices(graph, n_pad, chunk):
  n = graph.shape[0]
  g = jnp.pad(graph.astype(jnp.int32), ((0, n_pad - n), (0, 0)))
  return g.T.reshape(MAX_NB, n_pad // chunk, chunk).transpose(1, 0, 2)


def _tc_input_proj(fbonds, W_i):
  blk = 2000

  def body(x_ref, w_ref, b_ref, m_ref):
    b = jnp.dot(x_ref[...], w_ref[...], preferred_element_type=jnp.float32)
    b_ref[...] = b.astype(jnp.bfloat16)
    m_ref[...] = jnp.maximum(b, 0.0)

  return pl.pallas_call(
      body,
      grid=(N_BONDS // blk,),
      in_specs=[
          pl.BlockSpec((blk, IN_BOND), lambda i: (i, 0)),
          pl.BlockSpec((IN_BOND, HIDDEN), lambda i: (0, 0)),
      ],
      out_specs=[
          pl.BlockSpec((blk, HIDDEN), lambda i: (i, 0)),
          pl.BlockSpec((blk, HIDDEN), lambda i: (i, 0)),
      ],
      out_shape=[
          jax.ShapeDtypeStruct((N_BONDS, HIDDEN), jnp.bfloat16),
          jax.ShapeDtypeStruct((N_BONDS, HIDDEN), jnp.float32),
      ],
  )(fbonds, W_i)


def _tc_msg_update(binput, nei_padded, W_h):
  blk = 2000

  def body(b_ref, n_ref, w_ref, o_ref):
    o_ref[...] = jnp.maximum(
        b_ref[...].astype(jnp.float32)
        + jnp.dot(n_ref[...], w_ref[...], preferred_element_type=jnp.float32),
        0.0,
    )

  return pl.pallas_call(
      body,
      grid=(N_BONDS // blk,),
      in_specs=[
          pl.BlockSpec((blk, HIDDEN), lambda i: (i, 0)),
          pl.BlockSpec((blk, HIDDEN), lambda i: (i, 0)),
          pl.BlockSpec((HIDDEN, HIDDEN), lambda i: (0, 0)),
      ],
      out_specs=pl.BlockSpec((blk, HIDDEN), lambda i: (i, 0)),
      out_shape=jax.ShapeDtypeStruct((N_BONDS, HIDDEN), jnp.float32),
  )(binput, nei_padded, W_h)


def _tc_readout(fatoms, anei, W_oa, W_on, b_o, seg):
  blk = 1000
  mols_per_blk = blk // ATOMS_PER_MOL

  def body(fa_ref, an_ref, woa_ref, won_ref, bo_ref, seg_ref, o_ref):
    h = (
        jnp.dot(fa_ref[...], woa_ref[...], preferred_element_type=jnp.float32)
        + jnp.dot(an_ref[...], won_ref[...], preferred_element_type=jnp.float32)
        + bo_ref[...]
    )
    h = jnp.maximum(h, 0.0)
    o_ref[...] = jnp.dot(seg_ref[...], h, preferred_element_type=jnp.float32)

  return pl.pallas_call(
      body,
      grid=(N_ATOMS // blk,),
      in_specs=[
          pl.BlockSpec((blk, ATOM_FDIM), lambda i: (i, 0)),
          pl.BlockSpec((blk, HIDDEN), lambda i: (i, 0)),
          pl.BlockSpec((ATOM_FDIM, HIDDEN), lambda i: (0, 0)),
          pl.BlockSpec((HIDDEN, HIDDEN), lambda i: (0, 0)),
          pl.BlockSpec((1, HIDDEN), lambda i: (0, 0)),
          pl.BlockSpec((mols_per_blk, blk), lambda i: (0, 0)),
      ],
      out_specs=pl.BlockSpec((mols_per_blk, HIDDEN), lambda i: (i, 0)),
      out_shape=jax.ShapeDtypeStruct((N_MOLS, HIDDEN), jnp.float32),
  )(fatoms, anei, W_oa, W_on, b_o, seg)


BOND_CHUNK = 128
BOND_STEPS = 49
BOND_PAD = NUM_WORKERS * BOND_STEPS * BOND_CHUNK

ATOM_CHUNK = 112
ATOM_STEPS = 14
ATOM_PAD = NUM_WORKERS * ATOM_STEPS * ATOM_CHUNK

_bond_gather = _make_gather_sum(BOND_PAD, BOND_CHUNK, BOND_STEPS, N_BONDS)
_atom_gather = _make_gather_sum(ATOM_PAD, ATOM_CHUNK, ATOM_STEPS, N_BONDS)


def kernel(fatoms, fbonds, agraph, bgraph, W_i, W_h, W_o, b_o):
  bidx = _pad_indices(bgraph, BOND_PAD, BOND_CHUNK)
  aidx = _pad_indices(agraph, ATOM_PAD, ATOM_CHUNK)

  binput, message = _tc_input_proj(fbonds, W_i)

  for _ in range(DEPTH - 1):
    nei = _bond_gather(message, bidx)
    message = _tc_msg_update(binput, nei, W_h)

  anei = _atom_gather(message, aidx)

  W_oa = W_o[:ATOM_FDIM]
  W_on = W_o[ATOM_FDIM:]
  blk = 1000
  mols_per_blk = blk // ATOMS_PER_MOL
  seg = jnp.kron(
      jnp.eye(mols_per_blk, dtype=jnp.float32),
      jnp.full((1, ATOMS_PER_MOL), 1.0 / ATOMS_PER_MOL, dtype=jnp.float32),
  )
  return _tc_readout(fatoms, anei, W_oa, W_on, b_o.reshape(1, HIDDEN), seg)
